# baseline (device time: 32307 ns/iter reference)
import jax
import jax.numpy as jnp
from jax import lax
from jax.experimental import pallas as pl
from jax.experimental.pallas import tpu as pltpu

NC = 16


def kernel(A, B):
    m, k = A.shape
    _, n = B.shape
    mc = m // NC

    def body(
        a_hbm, b_hbm, out_hbm,
        a_vmem, b_vmem, acc, send_buf, comm_ref,
        in_sems, out_sems, send_sems, recv_sems,
    ):
        my_x = lax.axis_index("x")
        my_y = lax.axis_index("y")
        peer = (1 - my_x, my_y)

        barrier_sem = pltpu.get_barrier_semaphore()
        pl.semaphore_signal(
            barrier_sem, inc=1, device_id=peer,
            device_id_type=pl.DeviceIdType.MESH,
        )

        a_cp = pltpu.make_async_copy(a_hbm, a_vmem, in_sems.at[0])
        b_cp = pltpu.make_async_copy(b_hbm, b_vmem, in_sems.at[1])
        a_cp.start()
        b_cp.start()
        a_cp.wait()
        b_cp.wait()

        rdmas = []
        for c in range(NC):
            rows = pl.ds(c * mc, mc)
            part = jnp.dot(
                a_vmem[rows, :], b_vmem[:, :],
                preferred_element_type=jnp.float32,
            )
            acc[rows, :] = part
            send_buf[c, :, :] = part.astype(jnp.bfloat16)
            if c == 0:
                pl.semaphore_wait(barrier_sem, 1)
            rdma = pltpu.make_async_remote_copy(
                src_ref=send_buf.at[c],
                dst_ref=comm_ref.at[c],
                send_sem=send_sems.at[c],
                recv_sem=recv_sems.at[c],
                device_id=peer,
                device_id_type=pl.DeviceIdType.MESH,
            )
            rdma.start()
            rdmas.append(rdma)

        out_cps = []
        for c in range(NC):
            rows = pl.ds(c * mc, mc)
            rdmas[c].wait_recv()
            acc[rows, :] = acc[rows, :] + comm_ref[c, :, :].astype(jnp.float32)
            cp = pltpu.make_async_copy(
                acc.at[rows, :], out_hbm.at[rows, :], out_sems.at[c]
            )
            cp.start()
            out_cps.append(cp)

        for c in range(NC):
            out_cps[c].wait()
            rdmas[c].wait_send()

    return pl.pallas_call(
        body,
        out_shape=jax.ShapeDtypeStruct((m, n), jnp.float32),
        in_specs=[
            pl.BlockSpec(memory_space=pl.ANY),
            pl.BlockSpec(memory_space=pl.ANY),
        ],
        out_specs=pl.BlockSpec(memory_space=pl.ANY),
        scratch_shapes=[
            pltpu.VMEM((m, k), jnp.float32),
            pltpu.VMEM((k, n), jnp.float32),
            pltpu.VMEM((m, n), jnp.float32),
            pltpu.VMEM((NC, mc, n), jnp.bfloat16),
            pltpu.VMEM((NC, mc, n), jnp.bfloat16),
            pltpu.SemaphoreType.DMA((2,)),
            pltpu.SemaphoreType.DMA((NC,)),
            pltpu.SemaphoreType.DMA((NC,)),
            pltpu.SemaphoreType.DMA((NC,)),
        ],
        compiler_params=pltpu.CompilerParams(collective_id=0),
    )(A, B)
